# baseline (speedup 1.0000x reference)
import functools

import jax
import jax.numpy as jnp
from jax import lax
from jax.experimental import pallas as pl
from jax.experimental.pallas import tpu as pltpu
from jax.experimental.pallas import tpu_sc as plsc

F32 = jnp.float32
I32 = jnp.int32

NC = 2
NSC = 16
NW = NC * NSC

CH = 128
CPW = 40
TPW = CH * CPW
SCH = 40
SCPW = TPW // SCH
T_PAD = TPW * NW
O_PAD = 10112
OPT = O_PAD // NSC
HID = 128
EBLK = 2048
NEBLK = T_PAD // EBLK
OBLK = 1264
NOBLK = O_PAD // OBLK


def _sc_mesh():
    return plsc.VectorSubcoreMesh(
        core_axis_name="c", subcore_axis_name="s", num_cores=NC,
        num_subcores=NSC)


def _sc_gather(u, v, sidx3, oidx3):
    def body(u_hbm, v_hbm, s_hbm, o_hbm, e_hbm, f_hbm,
             sidx_v, oidx_v,
             bufe0, buff0, bufe1, buff1,
             ge0, gf0, ge1, gf1, we0, wf0, we1, wf1):
        wid = lax.axis_index("s") * NC + lax.axis_index("c")
        base = wid * TPW
        pltpu.sync_copy(s_hbm.at[wid], sidx_v)
        pltpu.sync_copy(o_hbm.at[wid], oidx_v)

        def gat(j, be, bf, se, sf):
            pltpu.async_copy(u_hbm.at[sidx_v.at[j]], be, se)
            pltpu.async_copy(v_hbm.at[oidx_v.at[j]], bf, sf)

        def wait_gat(be, bf, se, sf):
            pltpu.make_async_copy(u_hbm.at[sidx_v.at[0]], be, se).wait()
            pltpu.make_async_copy(v_hbm.at[oidx_v.at[0]], bf, sf).wait()

        def wr(j, be, bf, se, sf):
            row = base + j * CH
            pltpu.async_copy(be, e_hbm.at[pl.ds(row, CH)], se)
            pltpu.async_copy(bf, f_hbm.at[pl.ds(row, CH)], sf)

        def wait_wr(be, bf, se, sf):
            pltpu.make_async_copy(be, e_hbm.at[pl.ds(base, CH)], se).wait()
            pltpu.make_async_copy(bf, f_hbm.at[pl.ds(base, CH)], sf).wait()

        gat(0, bufe0, buff0, ge0, gf0)
        gat(1, bufe1, buff1, ge1, gf1)

        def step(k, carry):
            j0 = 2 * k
            j1 = j0 + 1
            wait_gat(bufe0, buff0, ge0, gf0)
            wr(j0, bufe0, buff0, we0, wf0)
            wait_gat(bufe1, buff1, ge1, gf1)
            wr(j1, bufe1, buff1, we1, wf1)
            wait_wr(bufe0, buff0, we0, wf0)
            gat(j0 + 2, bufe0, buff0, ge0, gf0)
            wait_wr(bufe1, buff1, we1, wf1)
            gat(j1 + 2, bufe1, buff1, ge1, gf1)
            return carry

        lax.fori_loop(0, CPW // 2 - 1, step, 0)
        wait_gat(bufe0, buff0, ge0, gf0)
        wr(CPW - 2, bufe0, buff0, we0, wf0)
        wait_gat(bufe1, buff1, ge1, gf1)
        wr(CPW - 1, bufe1, buff1, we1, wf1)
        wait_wr(bufe0, buff0, we0, wf0)
        wait_wr(bufe1, buff1, we1, wf1)

    out_type = [jax.ShapeDtypeStruct((T_PAD, HID), F32),
                jax.ShapeDtypeStruct((T_PAD, HID), F32)]
    scratch = ([pltpu.VMEM((CPW, CH), I32), pltpu.VMEM((CPW, CH), I32)]
               + [pltpu.VMEM((CH, HID), F32)] * 4
               + [pltpu.SemaphoreType.DMA] * 8)
    fn = pl.kernel(body, out_type=out_type, mesh=_sc_mesh(),
                   scratch_types=scratch, name="sc_gather")
    return fn(u, v, sidx3, oidx3)


def _sc_scatter(ns, no, sidx3, oidx3, zeros_nd):
    def body(ns_hbm, no_hbm, s_hbm, o_hbm, z_hbm, pooled_hbm,
             sidx_v, oidx_v, bufs, bufo, acc,
             is0, io0, cs0, co0):
        c = lax.axis_index("c")
        sid = lax.axis_index("s")
        wid = sid * NC + c
        base = wid * TPW
        pltpu.sync_copy(z_hbm, acc.at[pl.ds(sid * OPT, OPT)])
        pltpu.sync_copy(s_hbm.at[wid], sidx_v)
        pltpu.sync_copy(o_hbm.at[wid], oidx_v)
        plsc.subcore_barrier()

        def rd_s(j):
            pltpu.async_copy(ns_hbm.at[pl.ds(base + j * CH, CH)], bufs, is0)

        def rd_o(j):
            pltpu.async_copy(no_hbm.at[pl.ds(base + j * CH, CH)], bufo, io0)

        def wait_rd_s():
            pltpu.make_async_copy(
                ns_hbm.at[pl.ds(base, CH)], bufs, is0).wait()

        def wait_rd_o():
            pltpu.make_async_copy(
                no_hbm.at[pl.ds(base, CH)], bufo, io0).wait()

        def sc_s(j):
            pltpu.async_copy(bufs, acc.at[sidx_v.at[j]], cs0, add=True)

        def sc_o(j):
            pltpu.async_copy(bufo, acc.at[oidx_v.at[j]], co0, add=True)

        def wait_sc_s():
            pltpu.make_async_copy(bufs, acc.at[sidx_v.at[0]], cs0).wait()

        def wait_sc_o():
            pltpu.make_async_copy(bufo, acc.at[oidx_v.at[0]], co0).wait()

        rd_s(0)
        rd_o(0)

        def step(j, carry):
            wait_rd_s()
            sc_s(j)
            wait_rd_o()
            sc_o(j)
            wait_sc_s()
            rd_s(j + 1)
            wait_sc_o()
            rd_o(j + 1)
            return carry

        lax.fori_loop(0, CPW - 1, step, 0)
        wait_rd_s()
        sc_s(CPW - 1)
        wait_rd_o()
        sc_o(CPW - 1)
        wait_sc_s()
        wait_sc_o()
        plsc.subcore_barrier()

        @pl.when(sid == 0)
        def _():
            pltpu.sync_copy(acc, pooled_hbm.at[c])

    out_type = [jax.ShapeDtypeStruct((NC, O_PAD, HID), F32)]
    scratch = ([pltpu.VMEM((CPW, CH), I32), pltpu.VMEM((CPW, CH), I32)]
               + [pltpu.VMEM((CH, HID), F32)] * 2
               + [pltpu.VMEM_SHARED((O_PAD, HID), F32)]
               + [pltpu.SemaphoreType.DMA] * 4)
    fn = pl.kernel(body, out_type=out_type, mesh=_sc_mesh(),
                   scratch_types=scratch, name="sc_scatter")
    return fn(ns, no, sidx3, oidx3, zeros_nd)


def _sc_counts(sidx3, oidx3, zeros_nd, ones_nd):
    def body(s_hbm, o_hbm, z_hbm, one_hbm, cnt_hbm,
             sidx_v, oidx_v, ones_v, acc, sem):
        c = lax.axis_index("c")
        sid = lax.axis_index("s")
        wid = sid * NC + c
        pltpu.sync_copy(z_hbm, acc.at[pl.ds(sid * OPT, OPT)])
        pltpu.sync_copy(one_hbm, ones_v)
        pltpu.sync_copy(s_hbm.at[wid], sidx_v)
        pltpu.sync_copy(o_hbm.at[wid], oidx_v)
        plsc.subcore_barrier()

        def chunk(j, carry):
            pltpu.sync_copy(ones_v, acc.at[sidx_v.at[j]], add=True)
            pltpu.sync_copy(ones_v, acc.at[oidx_v.at[j]], add=True)
            return carry

        lax.fori_loop(0, CPW, chunk, 0)
        plsc.subcore_barrier()

        @pl.when(sid == 0)
        def _():
            pltpu.sync_copy(acc, cnt_hbm.at[c])

    out_type = [jax.ShapeDtypeStruct((NC, O_PAD, HID), F32)]
    scratch = [pltpu.VMEM((CPW, CH), I32), pltpu.VMEM((CPW, CH), I32),
               pltpu.VMEM((CH, HID), F32),
               pltpu.VMEM_SHARED((O_PAD, HID), F32),
               pltpu.SemaphoreType.DMA]
    fn = pl.kernel(body, out_type=out_type, mesh=_sc_mesh(),
                   scratch_types=scratch, name="sc_counts")
    return fn(sidx3, oidx3, zeros_nd, ones_nd)


def _tc_prep0(objs3, tes0, teo0):
    nemb = tes0.shape[0]

    def body(objs_ref, tes_ref, teo_ref, u_ref, v_ref):
        idx = objs_ref[0, 0, :]
        oh = (idx[:, None] ==
              lax.broadcasted_iota(I32, (OBLK, nemb), 1)).astype(F32)
        u_ref[...] = jnp.dot(oh, tes_ref[...], preferred_element_type=F32)
        v_ref[...] = jnp.dot(oh, teo_ref[...], preferred_element_type=F32)

    return pl.pallas_call(
        body,
        grid=(NOBLK,),
        in_specs=[
            pl.BlockSpec((1, 1, OBLK), lambda i: (i, 0, 0)),
            pl.BlockSpec((nemb, HID), lambda i: (0, 0)),
            pl.BlockSpec((nemb, HID), lambda i: (0, 0)),
        ],
        out_specs=[
            pl.BlockSpec((OBLK, HID), lambda i: (i, 0)),
            pl.BlockSpec((OBLK, HID), lambda i: (i, 0)),
        ],
        out_shape=[jax.ShapeDtypeStruct((O_PAD, HID), F32),
                   jax.ShapeDtypeStruct((O_PAD, HID), F32)],
    )(objs3, tes0, teo0)


def _tc_edge(e, f, p_in, w1b, b1b, ap_next, pe0=None, p_idx3=None):
    first = pe0 is not None
    last = ap_next is None
    wcols = w1b.shape[1]

    def body(*refs):
        if first:
            e_ref, f_ref, pi_ref, pe_ref, w_ref, b_ref = refs[:6]
            rest = refs[6:]
        else:
            e_ref, f_ref, p_ref, w_ref, b_ref = refs[:5]
            rest = refs[5:]
        if last:
            ns_ref, no_ref = rest
        else:
            ap_ref = rest[0]
            ns_ref, no_ref, pn_ref = rest[1:]

        if first:
            pidx = pi_ref[0, 0, :]
            npred = pe_ref.shape[0]
            oh = (pidx[:, None] ==
                  lax.broadcasted_iota(I32, (EBLK, npred), 1)).astype(F32)
            p = jnp.dot(oh, pe_ref[...], preferred_element_type=F32)
        else:
            p = p_ref[...]
        h = jnp.maximum(e_ref[...] + f_ref[...] + p, 0.0)
        hb = h.astype(jnp.bfloat16)
        hl = (h - hb.astype(F32)).astype(jnp.bfloat16)
        w = w_ref[...].astype(jnp.bfloat16)
        t = (jnp.dot(hb, w, preferred_element_type=F32)
             + jnp.dot(hl, w, preferred_element_type=F32)) + b_ref[...]
        t = jnp.maximum(t, 0.0)
        ns_ref[...] = t[:, :HID]
        if last:
            no_ref[...] = t[:, HID:2 * HID]
        else:
            no_ref[...] = t[:, 2 * HID:3 * HID]
            tp = t[:, HID:2 * HID]
            tpb = tp.astype(jnp.bfloat16)
            tpl = (tp - tpb.astype(F32)).astype(jnp.bfloat16)
            ap = ap_ref[...].astype(jnp.bfloat16)
            pn_ref[...] = (jnp.dot(tpb, ap, preferred_element_type=F32)
                           + jnp.dot(tpl, ap, preferred_element_type=F32))

    eblk2 = pl.BlockSpec((EBLK, HID), lambda i: (i, 0))
    in_specs = [eblk2, eblk2]
    args = [e, f]
    if first:
        npred = pe0.shape[0]
        in_specs += [pl.BlockSpec((1, 1, EBLK), lambda i: (i, 0, 0)),
                     pl.BlockSpec((npred, HID), lambda i: (0, 0))]
        args += [p_idx3, pe0]
    else:
        in_specs += [eblk2]
        args += [p_in]
    in_specs += [pl.BlockSpec((HID, wcols), lambda i: (0, 0)),
                 pl.BlockSpec((1, wcols), lambda i: (0, 0))]
    args += [w1b, b1b]
    if not last:
        in_specs += [pl.BlockSpec((HID, HID), lambda i: (0, 0))]
        args += [ap_next]

    nout = 2 if last else 3
    out_specs = [eblk2] * nout
    out_shape = [jax.ShapeDtypeStruct((T_PAD, HID), F32)] * nout

    return pl.pallas_call(
        body, grid=(NEBLK,), in_specs=in_specs, out_specs=out_specs,
        out_shape=out_shape)(*args)


def _tc_node(pooledp, cntp, w2a, b2a, w2b, b2b, as_n, b1a_n, ao_n):
    def body(pp_ref, c_ref, w2a_ref, b2a_ref, w2b_ref, b2b_ref,
             as_ref, b1a_ref, ao_ref, u_ref, v_ref):
        cnt = c_ref[0, :, 0:1] + c_ref[1, :, 0:1]
        inv = 1.0 / jnp.maximum(cnt, 1.0)
        pooled = (pp_ref[0] + pp_ref[1]) * inv
        h2 = jnp.maximum(
            jnp.dot(pooled, w2a_ref[...], preferred_element_type=F32)
            + b2a_ref[...], 0.0)
        obj = jnp.maximum(
            jnp.dot(h2, w2b_ref[...], preferred_element_type=F32)
            + b2b_ref[...], 0.0)
        u_ref[...] = jnp.dot(obj, as_ref[...],
                             preferred_element_type=F32) + b1a_ref[...]
        v_ref[...] = jnp.dot(obj, ao_ref[...], preferred_element_type=F32)

    wspec = pl.BlockSpec((HID, HID), lambda i: (0, 0))
    bspec = pl.BlockSpec((1, HID), lambda i: (0, 0))
    return pl.pallas_call(
        body,
        grid=(NOBLK,),
        in_specs=[
            pl.BlockSpec((NC, OBLK, HID), lambda i: (0, i, 0)),
            pl.BlockSpec((NC, OBLK, HID), lambda i: (0, i, 0)),
            wspec, bspec, wspec, bspec, wspec, bspec, wspec,
        ],
        out_specs=[pl.BlockSpec((OBLK, HID), lambda i: (i, 0))] * 2,
        out_shape=[jax.ShapeDtypeStruct((O_PAD, HID), F32)] * 2,
    )(pooledp, cntp, w2a, b2a, w2b, b2b, as_n, b1a_n, ao_n)


def _tc_final(pooledp, cntp, w2a, b2a, w2b, b2b, o_real):
    def body(pp_ref, c_ref, w2a_ref, b2a_ref, w2b_ref, b2b_ref, out_ref):
        cnt = c_ref[0, :, 0:1] + c_ref[1, :, 0:1]
        inv = 1.0 / jnp.maximum(cnt, 1.0)
        pooled = (pp_ref[0] + pp_ref[1]) * inv
        h2 = jnp.maximum(
            jnp.dot(pooled, w2a_ref[...], preferred_element_type=F32)
            + b2a_ref[...], 0.0)
        obj = jnp.maximum(
            jnp.dot(h2, w2b_ref[...], preferred_element_type=F32)
            + b2b_ref[...], 0.0)
        rows = lax.broadcasted_iota(I32, (O_PAD, HID), 0)
        objm = jnp.where(rows < o_real, obj, 0.0)
        out_ref[...] = jnp.sum(objm, axis=0, keepdims=True) / float(o_real)

    return pl.pallas_call(
        body,
        in_specs=[
            pl.BlockSpec((NC, O_PAD, HID), lambda: (0, 0, 0)),
            pl.BlockSpec((NC, O_PAD, HID), lambda: (0, 0, 0)),
            pl.BlockSpec((HID, HID), lambda: (0, 0)),
            pl.BlockSpec((1, HID), lambda: (0, 0)),
            pl.BlockSpec((HID, HID), lambda: (0, 0)),
            pl.BlockSpec((1, HID), lambda: (0, 0)),
        ],
        out_specs=pl.BlockSpec((1, HID), lambda: (0, 0)),
        out_shape=jax.ShapeDtypeStruct((1, HID), F32),
    )(pooledp, cntp, w2a, b2a, w2b, b2b)


def kernel(objs, triples, obj_emb, pred_emb, params):
    n_obj = objs.shape[0]
    n_tr = triples.shape[0]
    emb = obj_emb.shape[1]
    nlayers = len(params)

    s_idx = triples[:, 0].astype(I32)
    p_idx = triples[:, 1].astype(I32)
    o_idx = triples[:, 2].astype(I32)

    npad_t = T_PAD - n_tr
    pad_nodes = n_obj + 16 + (jnp.arange(npad_t, dtype=I32) % 64)
    s_pad = jnp.concatenate([s_idx, pad_nodes])
    o_pad = jnp.concatenate([o_idx, pad_nodes])
    p_pad = jnp.concatenate([p_idx, jnp.zeros((npad_t,), I32)])
    sidx3 = s_pad.reshape(NW, CPW, CH)
    oidx3 = o_pad.reshape(NW, CPW, CH)
    pidx3 = p_pad.reshape(NEBLK, 1, EBLK)

    objs_p = jnp.concatenate(
        [objs.astype(I32), jnp.zeros((O_PAD - n_obj,), I32)])
    objs3 = objs_p.reshape(NOBLK, 1, OBLK)

    packed = []
    for i, pr in enumerate(params):
        din = emb if i == 0 else HID
        w1a = pr["w1a"]
        packed.append({
            "as": w1a[:din], "ap": w1a[din:2 * din], "ao": w1a[2 * din:],
            "b1a": pr["b1a"][None, :],
            "w1b": pr["w1b"], "b1b": pr["b1b"][None, :],
            "w2a": pr["w2a"], "b2a": pr["b2a"][None, :],
            "w2b": pr["w2b"], "b2b": pr["b2b"][None, :],
        })

    nemb_p = ((obj_emb.shape[0] + 7) // 8) * 8
    obj_emb_p = jnp.concatenate(
        [obj_emb, jnp.zeros((nemb_p - obj_emb.shape[0], emb), F32)])
    tes0 = obj_emb_p @ packed[0]["as"] + packed[0]["b1a"]
    teo0 = obj_emb_p @ packed[0]["ao"]
    npred_p = ((pred_emb.shape[0] + 7) // 8) * 8
    pred_emb_p = jnp.concatenate(
        [pred_emb, jnp.zeros((npred_p - pred_emb.shape[0], emb), F32)])
    pe0 = pred_emb_p @ packed[0]["ap"]

    zeros_nd = jnp.zeros((OPT, HID), F32)
    ones_nd = jnp.ones((CH, HID), F32)

    cntp = _sc_counts(sidx3, oidx3, zeros_nd, ones_nd)[0]
    u, v = _tc_prep0(objs3, tes0, teo0)
    e, f = _sc_gather(u, v, sidx3, oidx3)
    ns, no, p_dense = _tc_edge(
        e, f, None, packed[0]["w1b"], packed[0]["b1b"], packed[1]["ap"],
        pe0=pe0, p_idx3=pidx3)
    pooledp = _sc_scatter(ns, no, sidx3, oidx3, zeros_nd)[0]

    for i in range(1, nlayers):
        pr = packed[i]
        u, v = _tc_node(pooledp, cntp,
                        packed[i - 1]["w2a"], packed[i - 1]["b2a"],
                        packed[i - 1]["w2b"], packed[i - 1]["b2b"],
                        pr["as"], pr["b1a"], pr["ao"])
        e, f = _sc_gather(u, v, sidx3, oidx3)
        if i < nlayers - 1:
            ns, no, p_dense = _tc_edge(
                e, f, p_dense, pr["w1b"], pr["b1b"], packed[i + 1]["ap"])
        else:
            w1bc = jnp.concatenate(
                [pr["w1b"][:, :HID], pr["w1b"][:, 2 * HID:]], axis=1)
            b1bc = jnp.concatenate(
                [pr["b1b"][:, :HID], pr["b1b"][:, 2 * HID:]], axis=1)
            ns, no = _tc_edge(e, f, p_dense, w1bc, b1bc, None)
        pooledp = _sc_scatter(ns, no, sidx3, oidx3, zeros_nd)[0]

    out = _tc_final(pooledp, cntp,
                    packed[-1]["w2a"], packed[-1]["b2a"],
                    packed[-1]["w2b"], packed[-1]["b2b"], n_obj)
    return out.reshape(HID)


# speedup vs baseline: 1.0401x; 1.0401x over previous
import functools

import jax
import jax.numpy as jnp
from jax import lax
from jax.experimental import pallas as pl
from jax.experimental.pallas import tpu as pltpu
from jax.experimental.pallas import tpu_sc as plsc

F32 = jnp.float32
I32 = jnp.int32

NC = 2
NSC = 16
NW = NC * NSC

CH = 128
CPW = 40
TPW = CH * CPW
SCH = 40
SCPW = TPW // SCH
T_PAD = TPW * NW
O_PAD = 10112
OPT = O_PAD // NSC
HID = 128
EBLK = 2048
NEBLK = T_PAD // EBLK
OBLK = 1264
NOBLK = O_PAD // OBLK


def _sc_mesh():
    return plsc.VectorSubcoreMesh(
        core_axis_name="c", subcore_axis_name="s", num_cores=NC,
        num_subcores=NSC)


def _sc_gather(u, v, sidx3, oidx3):
    def body(u_hbm, v_hbm, s_hbm, o_hbm, e_hbm, f_hbm,
             sidx_v, oidx_v,
             bufe0, buff0, bufe1, buff1,
             ge0, gf0, ge1, gf1, we0, wf0, we1, wf1):
        wid = lax.axis_index("s") * NC + lax.axis_index("c")
        base = wid * TPW
        pltpu.sync_copy(s_hbm.at[wid], sidx_v)
        pltpu.sync_copy(o_hbm.at[wid], oidx_v)

        def gat(j, be, bf, se, sf):
            pltpu.async_copy(u_hbm.at[sidx_v.at[j]], be, se)
            pltpu.async_copy(v_hbm.at[oidx_v.at[j]], bf, sf)

        def wait_gat(be, bf, se, sf):
            pltpu.make_async_copy(u_hbm.at[sidx_v.at[0]], be, se).wait()
            pltpu.make_async_copy(v_hbm.at[oidx_v.at[0]], bf, sf).wait()

        def wr(j, be, bf, se, sf):
            row = base + j * CH
            pltpu.async_copy(be, e_hbm.at[pl.ds(row, CH)], se)
            pltpu.async_copy(bf, f_hbm.at[pl.ds(row, CH)], sf)

        def wait_wr(be, bf, se, sf):
            pltpu.make_async_copy(be, e_hbm.at[pl.ds(base, CH)], se).wait()
            pltpu.make_async_copy(bf, f_hbm.at[pl.ds(base, CH)], sf).wait()

        gat(0, bufe0, buff0, ge0, gf0)
        gat(1, bufe1, buff1, ge1, gf1)

        def step(k, carry):
            j0 = 2 * k
            j1 = j0 + 1
            wait_gat(bufe0, buff0, ge0, gf0)
            wr(j0, bufe0, buff0, we0, wf0)
            wait_gat(bufe1, buff1, ge1, gf1)
            wr(j1, bufe1, buff1, we1, wf1)
            wait_wr(bufe0, buff0, we0, wf0)
            gat(j0 + 2, bufe0, buff0, ge0, gf0)
            wait_wr(bufe1, buff1, we1, wf1)
            gat(j1 + 2, bufe1, buff1, ge1, gf1)
            return carry

        lax.fori_loop(0, CPW // 2 - 1, step, 0)
        wait_gat(bufe0, buff0, ge0, gf0)
        wr(CPW - 2, bufe0, buff0, we0, wf0)
        wait_gat(bufe1, buff1, ge1, gf1)
        wr(CPW - 1, bufe1, buff1, we1, wf1)
        wait_wr(bufe0, buff0, we0, wf0)
        wait_wr(bufe1, buff1, we1, wf1)

    out_type = [jax.ShapeDtypeStruct((T_PAD, HID), F32),
                jax.ShapeDtypeStruct((T_PAD, HID), F32)]
    scratch = ([pltpu.VMEM((CPW, CH), I32), pltpu.VMEM((CPW, CH), I32)]
               + [pltpu.VMEM((CH, HID), F32)] * 4
               + [pltpu.SemaphoreType.DMA] * 8)
    fn = pl.kernel(body, out_type=out_type, mesh=_sc_mesh(),
                   scratch_types=scratch, name="sc_gather")
    return fn(u, v, sidx3, oidx3)


def _sc_scatter(ns, no, sidx3, oidx3, zeros_nd):
    def body(ns_hbm, no_hbm, s_hbm, o_hbm, z_hbm, pooled_hbm,
             sidx_v, oidx_v, bufs, bufo, acc,
             is0, io0, cs0, co0):
        c = lax.axis_index("c")
        sid = lax.axis_index("s")
        wid = sid * NC + c
        base = wid * TPW
        pltpu.sync_copy(z_hbm, acc.at[pl.ds(sid * OPT, OPT)])
        pltpu.sync_copy(s_hbm.at[wid], sidx_v)
        pltpu.sync_copy(o_hbm.at[wid], oidx_v)
        plsc.subcore_barrier()

        def rd_s(j):
            pltpu.async_copy(ns_hbm.at[pl.ds(base + j * CH, CH)], bufs, is0)

        def rd_o(j):
            pltpu.async_copy(no_hbm.at[pl.ds(base + j * CH, CH)], bufo, io0)

        def wait_rd_s():
            pltpu.make_async_copy(
                ns_hbm.at[pl.ds(base, CH)], bufs, is0).wait()

        def wait_rd_o():
            pltpu.make_async_copy(
                no_hbm.at[pl.ds(base, CH)], bufo, io0).wait()

        def sc_s(j):
            pltpu.async_copy(bufs, acc.at[sidx_v.at[j]], cs0, add=True)

        def sc_o(j):
            pltpu.async_copy(bufo, acc.at[oidx_v.at[j]], co0, add=True)

        def wait_sc_s():
            pltpu.make_async_copy(bufs, acc.at[sidx_v.at[0]], cs0).wait()

        def wait_sc_o():
            pltpu.make_async_copy(bufo, acc.at[oidx_v.at[0]], co0).wait()

        rd_s(0)
        rd_o(0)

        def step(j, carry):
            wait_rd_s()
            sc_s(j)
            wait_rd_o()
            sc_o(j)
            wait_sc_s()
            rd_s(j + 1)
            wait_sc_o()
            rd_o(j + 1)
            return carry

        lax.fori_loop(0, CPW - 1, step, 0)
        wait_rd_s()
        sc_s(CPW - 1)
        wait_rd_o()
        sc_o(CPW - 1)
        wait_sc_s()
        wait_sc_o()
        plsc.subcore_barrier()

        @pl.when(sid == 0)
        def _():
            pltpu.sync_copy(acc, pooled_hbm.at[c])

    out_type = [jax.ShapeDtypeStruct((NC, O_PAD, HID), F32)]
    scratch = ([pltpu.VMEM((CPW, CH), I32), pltpu.VMEM((CPW, CH), I32)]
               + [pltpu.VMEM((CH, HID), F32)] * 2
               + [pltpu.VMEM_SHARED((O_PAD, HID), F32)]
               + [pltpu.SemaphoreType.DMA] * 4)
    fn = pl.kernel(body, out_type=out_type, mesh=_sc_mesh(),
                   scratch_types=scratch, name="sc_scatter")
    return fn(ns, no, sidx3, oidx3, zeros_nd)


def _sc_counts(sidx3, oidx3, zeros_nd, ones_nd):
    def body(s_hbm, o_hbm, z_hbm, one_hbm, cnt_hbm,
             sidx_v, oidx_v, ones_v, acc, sem):
        c = lax.axis_index("c")
        sid = lax.axis_index("s")
        wid = sid * NC + c
        pltpu.sync_copy(z_hbm, acc.at[pl.ds(sid * OPT, OPT)])
        pltpu.sync_copy(one_hbm, ones_v)
        pltpu.sync_copy(s_hbm.at[wid], sidx_v)
        pltpu.sync_copy(o_hbm.at[wid], oidx_v)
        plsc.subcore_barrier()

        def chunk(j, carry):
            pltpu.sync_copy(ones_v, acc.at[sidx_v.at[j]], add=True)
            pltpu.sync_copy(ones_v, acc.at[oidx_v.at[j]], add=True)
            return carry

        lax.fori_loop(0, CPW, chunk, 0)
        plsc.subcore_barrier()

        @pl.when(sid == 0)
        def _():
            pltpu.sync_copy(acc, cnt_hbm.at[c])

    out_type = [jax.ShapeDtypeStruct((NC, O_PAD, HID), F32)]
    scratch = [pltpu.VMEM((CPW, CH), I32), pltpu.VMEM((CPW, CH), I32),
               pltpu.VMEM((CH, HID), F32),
               pltpu.VMEM_SHARED((O_PAD, HID), F32),
               pltpu.SemaphoreType.DMA]
    fn = pl.kernel(body, out_type=out_type, mesh=_sc_mesh(),
                   scratch_types=scratch, name="sc_counts")
    return fn(sidx3, oidx3, zeros_nd, ones_nd)


def _tc_prep0(objs3, tes0, teo0):
    nemb = tes0.shape[0]

    def body(objs_ref, tes_ref, teo_ref, u_ref, v_ref):
        idx = objs_ref[0, 0, :]
        oh = (idx[:, None] ==
              lax.broadcasted_iota(I32, (OBLK, nemb), 1)).astype(F32)
        u_ref[...] = jnp.dot(oh, tes_ref[...], preferred_element_type=F32)
        v_ref[...] = jnp.dot(oh, teo_ref[...], preferred_element_type=F32)

    return pl.pallas_call(
        body,
        grid=(NOBLK,),
        in_specs=[
            pl.BlockSpec((1, 1, OBLK), lambda i: (i, 0, 0)),
            pl.BlockSpec((nemb, HID), lambda i: (0, 0)),
            pl.BlockSpec((nemb, HID), lambda i: (0, 0)),
        ],
        out_specs=[
            pl.BlockSpec((OBLK, HID), lambda i: (i, 0)),
            pl.BlockSpec((OBLK, HID), lambda i: (i, 0)),
        ],
        out_shape=[jax.ShapeDtypeStruct((O_PAD, HID), F32),
                   jax.ShapeDtypeStruct((O_PAD, HID), F32)],
    )(objs3, tes0, teo0)


def _tc_edge(e, f, p_in, w1b, b1b, ap_next, pe0=None, p_idx3=None):
    first = pe0 is not None
    last = ap_next is None
    wcols = w1b.shape[1]

    def body(*refs):
        if first:
            e_ref, f_ref, pi_ref, pe_ref, w_ref, b_ref = refs[:6]
            rest = refs[6:]
        else:
            e_ref, f_ref, p_ref, w_ref, b_ref = refs[:5]
            rest = refs[5:]
        if last:
            ns_ref, no_ref = rest
        else:
            ap_ref = rest[0]
            ns_ref, no_ref, pn_ref = rest[1:]

        if first:
            pidx = pi_ref[0, 0, :]
            npred = pe_ref.shape[0]
            oh = (pidx[:, None] ==
                  lax.broadcasted_iota(I32, (EBLK, npred), 1)).astype(F32)
            p = jnp.dot(oh, pe_ref[...], preferred_element_type=F32)
        else:
            p = p_ref[...]
        h = jnp.maximum(e_ref[...] + f_ref[...] + p, 0.0)
        hb = h.astype(jnp.bfloat16)
        w = w_ref[...].astype(jnp.bfloat16)
        t = jnp.dot(hb, w, preferred_element_type=F32) + b_ref[...]
        t = jnp.maximum(t, 0.0)
        ns_ref[...] = t[:, :HID]
        if last:
            no_ref[...] = t[:, HID:2 * HID]
        else:
            no_ref[...] = t[:, 2 * HID:3 * HID]
            tp = t[:, HID:2 * HID].astype(jnp.bfloat16)
            ap = ap_ref[...].astype(jnp.bfloat16)
            pn_ref[...] = jnp.dot(tp, ap, preferred_element_type=F32)

    eblk2 = pl.BlockSpec((EBLK, HID), lambda i: (i, 0))
    in_specs = [eblk2, eblk2]
    args = [e, f]
    if first:
        npred = pe0.shape[0]
        in_specs += [pl.BlockSpec((1, 1, EBLK), lambda i: (i, 0, 0)),
                     pl.BlockSpec((npred, HID), lambda i: (0, 0))]
        args += [p_idx3, pe0]
    else:
        in_specs += [eblk2]
        args += [p_in]
    in_specs += [pl.BlockSpec((HID, wcols), lambda i: (0, 0)),
                 pl.BlockSpec((1, wcols), lambda i: (0, 0))]
    args += [w1b, b1b]
    if not last:
        in_specs += [pl.BlockSpec((HID, HID), lambda i: (0, 0))]
        args += [ap_next]

    nout = 2 if last else 3
    out_specs = [eblk2] * nout
    out_shape = [jax.ShapeDtypeStruct((T_PAD, HID), F32)] * nout

    return pl.pallas_call(
        body, grid=(NEBLK,), in_specs=in_specs, out_specs=out_specs,
        out_shape=out_shape)(*args)


def _tc_node(pooledp, cntp, w2a, b2a, w2b, b2b, as_n, b1a_n, ao_n):
    def body(pp_ref, c_ref, w2a_ref, b2a_ref, w2b_ref, b2b_ref,
             as_ref, b1a_ref, ao_ref, u_ref, v_ref):
        cnt = c_ref[0, :, 0:1] + c_ref[1, :, 0:1]
        inv = 1.0 / jnp.maximum(cnt, 1.0)
        pooled = (pp_ref[0] + pp_ref[1]) * inv
        h2 = jnp.maximum(
            jnp.dot(pooled, w2a_ref[...], preferred_element_type=F32)
            + b2a_ref[...], 0.0)
        obj = jnp.maximum(
            jnp.dot(h2, w2b_ref[...], preferred_element_type=F32)
            + b2b_ref[...], 0.0)
        u_ref[...] = jnp.dot(obj, as_ref[...],
                             preferred_element_type=F32) + b1a_ref[...]
        v_ref[...] = jnp.dot(obj, ao_ref[...], preferred_element_type=F32)

    wspec = pl.BlockSpec((HID, HID), lambda i: (0, 0))
    bspec = pl.BlockSpec((1, HID), lambda i: (0, 0))
    return pl.pallas_call(
        body,
        grid=(NOBLK,),
        in_specs=[
            pl.BlockSpec((NC, OBLK, HID), lambda i: (0, i, 0)),
            pl.BlockSpec((NC, OBLK, HID), lambda i: (0, i, 0)),
            wspec, bspec, wspec, bspec, wspec, bspec, wspec,
        ],
        out_specs=[pl.BlockSpec((OBLK, HID), lambda i: (i, 0))] * 2,
        out_shape=[jax.ShapeDtypeStruct((O_PAD, HID), F32)] * 2,
    )(pooledp, cntp, w2a, b2a, w2b, b2b, as_n, b1a_n, ao_n)


def _tc_final(pooledp, cntp, w2a, b2a, w2b, b2b, o_real):
    def body(pp_ref, c_ref, w2a_ref, b2a_ref, w2b_ref, b2b_ref, out_ref):
        cnt = c_ref[0, :, 0:1] + c_ref[1, :, 0:1]
        inv = 1.0 / jnp.maximum(cnt, 1.0)
        pooled = (pp_ref[0] + pp_ref[1]) * inv
        h2 = jnp.maximum(
            jnp.dot(pooled, w2a_ref[...], preferred_element_type=F32)
            + b2a_ref[...], 0.0)
        obj = jnp.maximum(
            jnp.dot(h2, w2b_ref[...], preferred_element_type=F32)
            + b2b_ref[...], 0.0)
        rows = lax.broadcasted_iota(I32, (O_PAD, HID), 0)
        objm = jnp.where(rows < o_real, obj, 0.0)
        out_ref[...] = jnp.sum(objm, axis=0, keepdims=True) / float(o_real)

    return pl.pallas_call(
        body,
        in_specs=[
            pl.BlockSpec((NC, O_PAD, HID), lambda: (0, 0, 0)),
            pl.BlockSpec((NC, O_PAD, HID), lambda: (0, 0, 0)),
            pl.BlockSpec((HID, HID), lambda: (0, 0)),
            pl.BlockSpec((1, HID), lambda: (0, 0)),
            pl.BlockSpec((HID, HID), lambda: (0, 0)),
            pl.BlockSpec((1, HID), lambda: (0, 0)),
        ],
        out_specs=pl.BlockSpec((1, HID), lambda: (0, 0)),
        out_shape=jax.ShapeDtypeStruct((1, HID), F32),
    )(pooledp, cntp, w2a, b2a, w2b, b2b)


def kernel(objs, triples, obj_emb, pred_emb, params):
    n_obj = objs.shape[0]
    n_tr = triples.shape[0]
    emb = obj_emb.shape[1]
    nlayers = len(params)

    s_idx = triples[:, 0].astype(I32)
    p_idx = triples[:, 1].astype(I32)
    o_idx = triples[:, 2].astype(I32)

    npad_t = T_PAD - n_tr
    pad_nodes = n_obj + 16 + (jnp.arange(npad_t, dtype=I32) % 64)
    s_pad = jnp.concatenate([s_idx, pad_nodes])
    o_pad = jnp.concatenate([o_idx, pad_nodes])
    p_pad = jnp.concatenate([p_idx, jnp.zeros((npad_t,), I32)])
    sidx3 = s_pad.reshape(NW, CPW, CH)
    oidx3 = o_pad.reshape(NW, CPW, CH)
    pidx3 = p_pad.reshape(NEBLK, 1, EBLK)

    objs_p = jnp.concatenate(
        [objs.astype(I32), jnp.zeros((O_PAD - n_obj,), I32)])
    objs3 = objs_p.reshape(NOBLK, 1, OBLK)

    packed = []
    for i, pr in enumerate(params):
        din = emb if i == 0 else HID
        w1a = pr["w1a"]
        packed.append({
            "as": w1a[:din], "ap": w1a[din:2 * din], "ao": w1a[2 * din:],
            "b1a": pr["b1a"][None, :],
            "w1b": pr["w1b"], "b1b": pr["b1b"][None, :],
            "w2a": pr["w2a"], "b2a": pr["b2a"][None, :],
            "w2b": pr["w2b"], "b2b": pr["b2b"][None, :],
        })

    nemb_p = ((obj_emb.shape[0] + 7) // 8) * 8
    obj_emb_p = jnp.concatenate(
        [obj_emb, jnp.zeros((nemb_p - obj_emb.shape[0], emb), F32)])
    tes0 = obj_emb_p @ packed[0]["as"] + packed[0]["b1a"]
    teo0 = obj_emb_p @ packed[0]["ao"]
    npred_p = ((pred_emb.shape[0] + 7) // 8) * 8
    pred_emb_p = jnp.concatenate(
        [pred_emb, jnp.zeros((npred_p - pred_emb.shape[0], emb), F32)])
    pe0 = pred_emb_p @ packed[0]["ap"]

    zeros_nd = jnp.zeros((OPT, HID), F32)
    ones_nd = jnp.ones((CH, HID), F32)

    cntp = _sc_counts(sidx3, oidx3, zeros_nd, ones_nd)[0]
    u, v = _tc_prep0(objs3, tes0, teo0)
    e, f = _sc_gather(u, v, sidx3, oidx3)
    ns, no, p_dense = _tc_edge(
        e, f, None, packed[0]["w1b"], packed[0]["b1b"], packed[1]["ap"],
        pe0=pe0, p_idx3=pidx3)
    pooledp = _sc_scatter(ns, no, sidx3, oidx3, zeros_nd)[0]

    for i in range(1, nlayers):
        pr = packed[i]
        u, v = _tc_node(pooledp, cntp,
                        packed[i - 1]["w2a"], packed[i - 1]["b2a"],
                        packed[i - 1]["w2b"], packed[i - 1]["b2b"],
                        pr["as"], pr["b1a"], pr["ao"])
        e, f = _sc_gather(u, v, sidx3, oidx3)
        if i < nlayers - 1:
            ns, no, p_dense = _tc_edge(
                e, f, p_dense, pr["w1b"], pr["b1b"], packed[i + 1]["ap"])
        else:
            w1bc = jnp.concatenate(
                [pr["w1b"][:, :HID], pr["w1b"][:, 2 * HID:]], axis=1)
            b1bc = jnp.concatenate(
                [pr["b1b"][:, :HID], pr["b1b"][:, 2 * HID:]], axis=1)
            ns, no = _tc_edge(e, f, p_dense, w1bc, b1bc, None)
        pooledp = _sc_scatter(ns, no, sidx3, oidx3, zeros_nd)[0]

    out = _tc_final(pooledp, cntp,
                    packed[-1]["w2a"], packed[-1]["b2a"],
                    packed[-1]["w2b"], packed[-1]["b2b"], n_obj)
    return out.reshape(HID)
